# Initial kernel scaffold; baseline (speedup 1.0000x reference)
import jax
import jax.numpy as jnp
from jax.experimental import pallas as pl


def kernel(features, edges, W0s, b0s, W1s, b1s, gammas, betas):
    raise NotImplementedError("write your pallas kernel here")


# speedup vs baseline: 3.6237x; 3.6237x over previous
import functools

import jax
import jax.numpy as jnp
from jax import lax
from jax.experimental import pallas as pl
from jax.experimental.pallas import tpu as pltpu
from jax.experimental.pallas import tpu_sc as plsc

N = 10000
D = 128
E = 320000
N_LAYERS = 3
EPS = 1e-5

NC = 2
NS = 16
E2 = 2 * E
PER_W = E2 // (NC * NS)
B = 80
NCHUNK = PER_W // B
ROWS_PER_SUB = 624


def _sc_agg_body(h1_hbm, gidx_hbm, sidx_hbm, zeros_hbm, out_hbm,
                 gidx_v, sidx_v, rows_v, agg_sh):
    c = lax.axis_index("c")
    s = lax.axis_index("s")
    base = (c * NS + s) * PER_W

    pltpu.sync_copy(zeros_hbm.at[pl.ds(s * ROWS_PER_SUB, ROWS_PER_SUB)],
                    agg_sh.at[pl.ds(s * ROWS_PER_SUB, ROWS_PER_SUB)])

    @pl.when(s == NS - 1)
    def _():
        pltpu.sync_copy(zeros_hbm.at[pl.ds(NS * ROWS_PER_SUB, N - NS * ROWS_PER_SUB)],
                        agg_sh.at[pl.ds(NS * ROWS_PER_SUB, N - NS * ROWS_PER_SUB)])

    plsc.subcore_barrier()

    @pl.loop(0, NCHUNK)
    def _(k):
        off = base + k * B
        pltpu.sync_copy(gidx_hbm.at[pl.ds(off, B)], gidx_v.at[0])
        pltpu.sync_copy(sidx_hbm.at[pl.ds(off, B)], sidx_v.at[0])
        pltpu.sync_copy(h1_hbm.at[gidx_v.at[0]], rows_v)
        pltpu.sync_copy(rows_v, agg_sh.at[sidx_v.at[0]], add=True)

    plsc.subcore_barrier()
    pltpu.sync_copy(agg_sh.at[pl.ds(s * ROWS_PER_SUB, ROWS_PER_SUB)],
                    out_hbm.at[c].at[pl.ds(s * ROWS_PER_SUB, ROWS_PER_SUB)])

    @pl.when(s == NS - 1)
    def _():
        pltpu.sync_copy(agg_sh.at[pl.ds(NS * ROWS_PER_SUB, N - NS * ROWS_PER_SUB)],
                        out_hbm.at[c].at[pl.ds(NS * ROWS_PER_SUB, N - NS * ROWS_PER_SUB)])


def _sc_agg(h1, gidx, sidx, zeros):
    kfn = pl.kernel(
        _sc_agg_body,
        out_type=jax.ShapeDtypeStruct((NC, N, D), jnp.float32),
        mesh=plsc.VectorSubcoreMesh(core_axis_name="c", subcore_axis_name="s"),
        scratch_types=[
            pltpu.VMEM((1, B), jnp.int32),
            pltpu.VMEM((1, B), jnp.int32),
            pltpu.VMEM((B, D), jnp.float32),
            pltpu.VMEM_SHARED((N, D), jnp.float32),
        ],
    )
    return kfn(h1, gidx, sidx, zeros)


BM = 1000


def _h1_body(f_ref, w_ref, b_ref, o_ref):
    o_ref[...] = (jnp.dot(f_ref[...], w_ref[...],
                          preferred_element_type=jnp.float32) + b_ref[...])


def _h1_matmul(f, W, b):
    return pl.pallas_call(
        _h1_body,
        grid=(N // BM,),
        in_specs=[pl.BlockSpec((BM, D), lambda i: (i, 0)),
                  pl.BlockSpec((D, D), lambda i: (0, 0)),
                  pl.BlockSpec((1, D), lambda i: (0, 0))],
        out_specs=pl.BlockSpec((BM, D), lambda i: (i, 0)),
        out_shape=jax.ShapeDtypeStruct((N, D), jnp.float32),
    )(f, W, b.reshape(1, D))


def _norm_body(f_ref, w_ref, b_ref, agg_ref, g_ref, bet_ref, *rest, add_res):
    if add_res:
        res_ref, o_ref = rest
    else:
        (o_ref,) = rest
    h = (jnp.dot(f_ref[...], w_ref[...], preferred_element_type=jnp.float32)
         + b_ref[...])
    h = h + agg_ref[0] + agg_ref[1]
    mu = jnp.mean(h, axis=-1, keepdims=True)
    var = jnp.mean((h - mu) ** 2, axis=-1, keepdims=True)
    h = (h - mu) * lax.rsqrt(var + EPS) * g_ref[...] + bet_ref[...]
    if add_res:
        h = h + res_ref[...]
    o_ref[...] = jnp.maximum(h, 0.0)


def _norm(f, W, b, agg, gamma, beta, res):
    add_res = res is not None
    in_specs = [pl.BlockSpec((BM, D), lambda i: (i, 0)),
                pl.BlockSpec((D, D), lambda i: (0, 0)),
                pl.BlockSpec((1, D), lambda i: (0, 0)),
                pl.BlockSpec((NC, BM, D), lambda i: (0, i, 0)),
                pl.BlockSpec((1, D), lambda i: (0, 0)),
                pl.BlockSpec((1, D), lambda i: (0, 0))]
    args = [f, W, b.reshape(1, D), agg, gamma.reshape(1, D),
            beta.reshape(1, D)]
    if add_res:
        in_specs.append(pl.BlockSpec((BM, D), lambda i: (i, 0)))
        args.append(res)
    return pl.pallas_call(
        functools.partial(_norm_body, add_res=add_res),
        grid=(N // BM,),
        in_specs=in_specs,
        out_specs=pl.BlockSpec((BM, D), lambda i: (i, 0)),
        out_shape=jax.ShapeDtypeStruct((N, D), jnp.float32),
    )(*args)


def kernel(features, edges, W0s, b0s, W1s, b1s, gammas, betas):
    src = edges[:, 0]
    dst = edges[:, 1]
    gidx = jnp.concatenate([dst, src])
    sidx = jnp.concatenate([src, dst])
    zeros = jnp.zeros((N, D), jnp.float32)
    f = features
    for i in range(N_LAYERS):
        h1 = _h1_matmul(f, W1s[i], b1s[i])
        agg = _sc_agg(h1, gidx, sidx, zeros)
        f = _norm(f, W0s[i], b0s[i], agg, gammas[i], betas[i],
                  features if i == N_LAYERS - 1 else None)
    return f


# speedup vs baseline: 8.9989x; 2.4833x over previous
import functools

import jax
import jax.numpy as jnp
from jax import lax
from jax.experimental import pallas as pl
from jax.experimental.pallas import tpu as pltpu
from jax.experimental.pallas import tpu_sc as plsc

N = 10000
D = 128
E = 320000
N_LAYERS = 3
EPS = 1e-5

NC = 2
NS = 16
B = 80
NCHUNK = E // (NC * NS * B)
NBLK = 5
BLK = NCHUNK // NBLK
ROWS_PER_SUB = 624


def _sc_agg_body(h1_hbm, src_hbm, dst_hbm, zeros_hbm, out_hbm,
                 src_v, dst_v, rows_v, agg_sh,
                 zsem, ssem0, ssem1, dsem0, dsem1, gsem0, gsem1):
    c = lax.axis_index("c")
    s = lax.axis_index("s")
    w = c * NS + s

    zc0 = pltpu.make_async_copy(
        zeros_hbm.at[pl.ds(s * ROWS_PER_SUB, ROWS_PER_SUB)],
        agg_sh.at[pl.ds(s * ROWS_PER_SUB, ROWS_PER_SUB)], zsem)
    zc0.start()

    @pl.when(s == NS - 1)
    def _():
        pltpu.sync_copy(
            zeros_hbm.at[pl.ds(NS * ROWS_PER_SUB, N - NS * ROWS_PER_SUB)],
            agg_sh.at[pl.ds(NS * ROWS_PER_SUB, N - NS * ROWS_PER_SUB)])

    ssems = (ssem0, ssem1)
    dsems = (dsem0, dsem1)

    def idx_copies(blk, ib):
        return (pltpu.make_async_copy(src_hbm.at[w, blk], src_v.at[ib],
                                      ssems[ib]),
                pltpu.make_async_copy(dst_hbm.at[w, blk], dst_v.at[ib],
                                      dsems[ib]))

    def gA(ib, k):
        return pltpu.make_async_copy(h1_hbm.at[dst_v.at[ib, k]],
                                     rows_v.at[0], gsem0)

    def gB(ib, k):
        return pltpu.make_async_copy(h1_hbm.at[src_v.at[ib, k]],
                                     rows_v.at[1], gsem1)

    def drainA(ib, k):
        gA(ib, k).wait()
        pltpu.sync_copy(rows_v.at[0], agg_sh.at[src_v.at[ib, k]], add=True)

    def drainB(ib, k):
        gB(ib, k).wait()
        pltpu.sync_copy(rows_v.at[1], agg_sh.at[dst_v.at[ib, k]], add=True)

    for cp in idx_copies(0, 0):
        cp.start()

    for blk in range(NBLK):
        ib = blk % 2
        for cp in idx_copies(blk, ib):
            cp.wait()
        if blk + 1 < NBLK:
            for cp in idx_copies(blk + 1, 1 - ib):
                cp.start()
        gA(ib, 0).start()
        gB(ib, 0).start()
        if blk == 0:
            zc0.wait()
            plsc.subcore_barrier()

        @pl.loop(0, BLK - 1)
        def _(k, ib=ib):
            drainA(ib, k)
            gA(ib, k + 1).start()
            drainB(ib, k)
            gB(ib, k + 1).start()

        drainA(ib, BLK - 1)
        drainB(ib, BLK - 1)

    plsc.subcore_barrier()
    pltpu.sync_copy(agg_sh.at[pl.ds(s * ROWS_PER_SUB, ROWS_PER_SUB)],
                    out_hbm.at[c].at[pl.ds(s * ROWS_PER_SUB, ROWS_PER_SUB)])

    @pl.when(s == NS - 1)
    def _():
        pltpu.sync_copy(agg_sh.at[pl.ds(NS * ROWS_PER_SUB, N - NS * ROWS_PER_SUB)],
                        out_hbm.at[c].at[pl.ds(NS * ROWS_PER_SUB, N - NS * ROWS_PER_SUB)])


def _sc_agg(h1, src3, dst3, zeros):
    kfn = pl.kernel(
        _sc_agg_body,
        out_type=jax.ShapeDtypeStruct((NC, N, D), jnp.float32),
        mesh=plsc.VectorSubcoreMesh(core_axis_name="c", subcore_axis_name="s"),
        scratch_types=[
            pltpu.VMEM((2, BLK, B), jnp.int32),
            pltpu.VMEM((2, BLK, B), jnp.int32),
            pltpu.VMEM((2, B, D), jnp.float32),
            pltpu.VMEM_SHARED((N, D), jnp.float32),
        ] + [pltpu.SemaphoreType.DMA] * 7,
    )
    return kfn(h1, src3, dst3, zeros)


BM = 1000


def _h1_body(f_ref, w_ref, b_ref, o_ref):
    o_ref[...] = (jnp.dot(f_ref[...], w_ref[...],
                          preferred_element_type=jnp.float32) + b_ref[...])


def _h1_matmul(f, W, b):
    return pl.pallas_call(
        _h1_body,
        grid=(N // BM,),
        in_specs=[pl.BlockSpec((BM, D), lambda i: (i, 0)),
                  pl.BlockSpec((D, D), lambda i: (0, 0)),
                  pl.BlockSpec((1, D), lambda i: (0, 0))],
        out_specs=pl.BlockSpec((BM, D), lambda i: (i, 0)),
        out_shape=jax.ShapeDtypeStruct((N, D), jnp.float32),
    )(f, W, b.reshape(1, D))


def _norm_body(f_ref, w_ref, b_ref, agg_ref, g_ref, bet_ref, *rest, add_res):
    if add_res:
        res_ref, o_ref = rest
    else:
        (o_ref,) = rest
    h = (jnp.dot(f_ref[...], w_ref[...], preferred_element_type=jnp.float32)
         + b_ref[...])
    h = h + agg_ref[0] + agg_ref[1]
    mu = jnp.mean(h, axis=-1, keepdims=True)
    var = jnp.mean((h - mu) ** 2, axis=-1, keepdims=True)
    h = (h - mu) * lax.rsqrt(var + EPS) * g_ref[...] + bet_ref[...]
    if add_res:
        h = h + res_ref[...]
    o_ref[...] = jnp.maximum(h, 0.0)


def _norm(f, W, b, agg, gamma, beta, res):
    add_res = res is not None
    in_specs = [pl.BlockSpec((BM, D), lambda i: (i, 0)),
                pl.BlockSpec((D, D), lambda i: (0, 0)),
                pl.BlockSpec((1, D), lambda i: (0, 0)),
                pl.BlockSpec((NC, BM, D), lambda i: (0, i, 0)),
                pl.BlockSpec((1, D), lambda i: (0, 0)),
                pl.BlockSpec((1, D), lambda i: (0, 0))]
    args = [f, W, b.reshape(1, D), agg, gamma.reshape(1, D),
            beta.reshape(1, D)]
    if add_res:
        in_specs.append(pl.BlockSpec((BM, D), lambda i: (i, 0)))
        args.append(res)
    return pl.pallas_call(
        functools.partial(_norm_body, add_res=add_res),
        grid=(N // BM,),
        in_specs=in_specs,
        out_specs=pl.BlockSpec((BM, D), lambda i: (i, 0)),
        out_shape=jax.ShapeDtypeStruct((N, D), jnp.float32),
    )(*args)


def kernel(features, edges, W0s, b0s, W1s, b1s, gammas, betas):
    src3 = edges[:, 0].reshape(NC * NS, NBLK, BLK, B)
    dst3 = edges[:, 1].reshape(NC * NS, NBLK, BLK, B)
    zeros = jnp.zeros((N, D), jnp.float32)
    f = features
    for i in range(N_LAYERS):
        h1 = _h1_matmul(f, W1s[i], b1s[i])
        agg = _sc_agg(h1, src3, dst3, zeros)
        f = _norm(f, W0s[i], b0s[i], agg, gammas[i], betas[i],
                  features if i == N_LAYERS - 1 else None)
    return f


# speedup vs baseline: 9.9898x; 1.1101x over previous
import functools

import jax
import jax.numpy as jnp
from jax import lax
from jax.experimental import pallas as pl
from jax.experimental.pallas import tpu as pltpu
from jax.experimental.pallas import tpu_sc as plsc

N = 10000
D = 128
E = 320000
N_LAYERS = 3
EPS = 1e-5

NC = 2
NS = 16
B = 40
NCHUNK = E // (NC * NS * B)
NBLK = 5
BLK = NCHUNK // NBLK
GRP = BLK // 2
ROWS_PER_SUB = 624


def _sc_agg_body(h1_hbm, src_hbm, dst_hbm, zeros_hbm, out_hbm,
                 src_v, dst_v, rows_v, agg_sh,
                 zsem, ssem0, ssem1, dsem0, dsem1,
                 g0, g1, g2, g3, c0, c1, c2, c3):
    c = lax.axis_index("c")
    s = lax.axis_index("s")
    w = c * NS + s
    gsems = (g0, g1, g2, g3)
    csems = (c0, c1, c2, c3)
    ssems = (ssem0, ssem1)
    dsems = (dsem0, dsem1)


    def idx_copies(blk, ibuf):
        return (pltpu.make_async_copy(src_hbm.at[w, blk], src_v.at[ibuf],
                                      ssems[ibuf]),
                pltpu.make_async_copy(dst_hbm.at[w, blk], dst_v.at[ibuf],
                                      dsems[ibuf]))

    def g_start(ibuf, ck, d, buf):
        idx = dst_v if d == 0 else src_v
        pltpu.make_async_copy(h1_hbm.at[idx.at[ibuf, ck]], rows_v.at[buf],
                              gsems[buf]).start()

    def g_wait(buf):
        pltpu.make_async_copy(h1_hbm.at[src_v.at[0, 0]], rows_v.at[buf],
                              gsems[buf]).wait()

    def sc_start(ibuf, ck, d, buf):
        idx = src_v if d == 0 else dst_v
        pltpu.make_async_copy(rows_v.at[buf], agg_sh.at[idx.at[ibuf, ck]],
                              csems[buf]).start(add=True)

    def sc_wait(buf):
        pltpu.make_async_copy(rows_v.at[buf], agg_sh.at[src_v.at[0, 0]],
                              csems[buf]).wait()

    def item(j, ibuf, m, skip_wait_sc=False, g_target=None):
        ck = 2 * j + (m >> 1)
        g_wait(m)
        sc_start(ibuf, ck, m & 1, m)
        if not skip_wait_sc:
            sc_wait((m - 1) % 4)
        if g_target is not None:
            ib3, ck3 = g_target
            g_start(ib3, ck3, (m + 3) & 1, (m + 3) % 4)

    zc0 = pltpu.make_async_copy(
        zeros_hbm.at[pl.ds(s * ROWS_PER_SUB, ROWS_PER_SUB)],
        agg_sh.at[pl.ds(s * ROWS_PER_SUB, ROWS_PER_SUB)], zsem)
    zc0.start()
    for cp in idx_copies(0, 0):
        cp.start()

    @pl.when(s == NS - 1)
    def _():
        pltpu.sync_copy(
            zeros_hbm.at[pl.ds(NS * ROWS_PER_SUB, N - NS * ROWS_PER_SUB)],
            agg_sh.at[pl.ds(NS * ROWS_PER_SUB, N - NS * ROWS_PER_SUB)])

    for cp in idx_copies(0, 0):
        cp.wait()
    for cp in idx_copies(1, 1):
        cp.start()
    g_start(0, 0, 0, 0)
    g_start(0, 0, 1, 1)
    g_start(0, 1, 0, 2)
    zc0.wait()
    plsc.subcore_barrier()

    for blk in range(NBLK):
        ib = blk % 2
        for m in range(4):
            item(0, ib, m, skip_wait_sc=(blk == 0 and m == 0),
                 g_target=(ib, (m + 3) >> 1))
        if 1 <= blk < NBLK - 1:
            for cp in idx_copies(blk + 1, 1 - ib):
                cp.start()

        @pl.loop(1, GRP - 1)
        def _(j, ib=ib):
            for m in range(4):
                item(j, ib, m, g_target=(ib, 2 * j + ((m + 3) >> 1)))

        jl = GRP - 1
        if blk + 1 < NBLK:
            nib = 1 - ib
            for cp in idx_copies(blk + 1, nib):
                cp.wait()
            item(jl, ib, 0, g_target=(ib, 2 * jl + 1))
            for m in (1, 2, 3):
                item(jl, ib, m, g_target=(nib, (m - 1) >> 1))
        else:
            item(jl, ib, 0, g_target=(ib, 2 * jl + 1))
            for m in (1, 2, 3):
                item(jl, ib, m, g_target=None)

    sc_wait(3)
    plsc.subcore_barrier()
    pltpu.sync_copy(agg_sh.at[pl.ds(s * ROWS_PER_SUB, ROWS_PER_SUB)],
                    out_hbm.at[c].at[pl.ds(s * ROWS_PER_SUB, ROWS_PER_SUB)])

    @pl.when(s == NS - 1)
    def _():
        pltpu.sync_copy(
            agg_sh.at[pl.ds(NS * ROWS_PER_SUB, N - NS * ROWS_PER_SUB)],
            out_hbm.at[c].at[pl.ds(NS * ROWS_PER_SUB, N - NS * ROWS_PER_SUB)])


def _sc_agg(h1, src4, dst4, zeros):
    kfn = pl.kernel(
        _sc_agg_body,
        out_type=jax.ShapeDtypeStruct((NC, N, D), jnp.float32),
        mesh=plsc.VectorSubcoreMesh(core_axis_name="c", subcore_axis_name="s"),
        scratch_types=[
            pltpu.VMEM((2, BLK, B), jnp.int32),
            pltpu.VMEM((2, BLK, B), jnp.int32),
            pltpu.VMEM((4, B, D), jnp.float32),
            pltpu.VMEM_SHARED((N, D), jnp.float32),
        ] + [pltpu.SemaphoreType.DMA] * 13,
    )
    return kfn(h1, src4, dst4, zeros)


BM = 1000


def _mm_body(f_ref, w_ref, b_ref, o_ref):
    o_ref[...] = (jnp.dot(f_ref[...], w_ref[...],
                          preferred_element_type=jnp.float32) + b_ref[...])


def _matmul(f, W, b):
    return pl.pallas_call(
        _mm_body,
        grid=(N // BM,),
        in_specs=[pl.BlockSpec((BM, D), lambda i: (i, 0)),
                  pl.BlockSpec((D, D), lambda i: (0, 0)),
                  pl.BlockSpec((1, D), lambda i: (0, 0))],
        out_specs=pl.BlockSpec((BM, D), lambda i: (i, 0)),
        out_shape=jax.ShapeDtypeStruct((N, D), jnp.float32),
    )(f, W, b.reshape(1, D))


def _norm_body(h0_ref, agg_ref, g_ref, bet_ref, *rest, add_res):
    if add_res:
        res_ref, o_ref = rest
    else:
        (o_ref,) = rest
    h = h0_ref[...] + agg_ref[0] + agg_ref[1]
    mu = jnp.mean(h, axis=-1, keepdims=True)
    var = jnp.mean((h - mu) ** 2, axis=-1, keepdims=True)
    h = (h - mu) * lax.rsqrt(var + EPS) * g_ref[...] + bet_ref[...]
    if add_res:
        h = h + res_ref[...]
    o_ref[...] = jnp.maximum(h, 0.0)


def _norm(h0, agg, gamma, beta, res):
    add_res = res is not None
    in_specs = [pl.BlockSpec((BM, D), lambda i: (i, 0)),
                pl.BlockSpec((NC, BM, D), lambda i: (0, i, 0)),
                pl.BlockSpec((1, D), lambda i: (0, 0)),
                pl.BlockSpec((1, D), lambda i: (0, 0))]
    args = [h0, agg, gamma.reshape(1, D), beta.reshape(1, D)]
    if add_res:
        in_specs.append(pl.BlockSpec((BM, D), lambda i: (i, 0)))
        args.append(res)
    return pl.pallas_call(
        functools.partial(_norm_body, add_res=add_res),
        grid=(N // BM,),
        in_specs=in_specs,
        out_specs=pl.BlockSpec((BM, D), lambda i: (i, 0)),
        out_shape=jax.ShapeDtypeStruct((N, D), jnp.float32),
    )(*args)


def kernel(features, edges, W0s, b0s, W1s, b1s, gammas, betas):
    src4 = edges[:, 0].reshape(NC * NS, NBLK, BLK, B)
    dst4 = edges[:, 1].reshape(NC * NS, NBLK, BLK, B)
    zeros = jnp.zeros((N, D), jnp.float32)
    f = features
    for i in range(N_LAYERS):
        h1 = _matmul(f, W1s[i], b1s[i])
        h0 = _matmul(f, W0s[i], b0s[i])
        agg = _sc_agg(h1, src4, dst4, zeros)
        f = _norm(h0, agg, gammas[i], betas[i],
                  features if i == N_LAYERS - 1 else None)
    return f


# speedup vs baseline: 10.6410x; 1.0652x over previous
import functools

import jax
import jax.numpy as jnp
from jax import lax
from jax.experimental import pallas as pl
from jax.experimental.pallas import tpu as pltpu
from jax.experimental.pallas import tpu_sc as plsc

N = 10000
D = 128
E = 320000
N_LAYERS = 3
EPS = 1e-5

NC = 2
NS = 16
B = 40
NCHUNK = E // (NC * NS * B)
NBLK = 10
BLK = NCHUNK // NBLK
NBUF = 5
UNROLL = 10
GRP = 2 * BLK // UNROLL
ROWS_PER_SUB = 624


def _sc_agg_body(h1_hbm, src_hbm, dst_hbm, zeros_hbm, out_hbm,
                 src_v, dst_v, rows_v, agg_sh,
                 zsem, ssem0, ssem1, dsem0, dsem1,
                 g0, g1, g2, g3, g4, c0, c1, c2, c3, c4):
    c = lax.axis_index("c")
    s = lax.axis_index("s")
    w = c * NS + s
    gsems = (g0, g1, g2, g3, g4)
    csems = (c0, c1, c2, c3, c4)
    ssems = (ssem0, ssem1)
    dsems = (dsem0, dsem1)


    def idx_copies(blk, ibuf):
        return (pltpu.make_async_copy(src_hbm.at[w, blk], src_v.at[ibuf],
                                      ssems[ibuf]),
                pltpu.make_async_copy(dst_hbm.at[w, blk], dst_v.at[ibuf],
                                      dsems[ibuf]))

    def g_start(ibuf, ck, d, buf):
        idx = dst_v if d == 0 else src_v
        pltpu.make_async_copy(h1_hbm.at[idx.at[ibuf, ck]], rows_v.at[buf],
                              gsems[buf]).start()

    def g_wait(buf):
        pltpu.make_async_copy(h1_hbm.at[src_v.at[0, 0]], rows_v.at[buf],
                              gsems[buf]).wait()

    def sc_start(ibuf, ck, d, buf):
        idx = src_v if d == 0 else dst_v
        pltpu.make_async_copy(rows_v.at[buf], agg_sh.at[idx.at[ibuf, ck]],
                              csems[buf]).start(add=True)

    def sc_wait(buf):
        pltpu.make_async_copy(rows_v.at[buf], agg_sh.at[src_v.at[0, 0]],
                              csems[buf]).wait()

    def item(j, ibuf, m, skip_wait_sc=False, g_target=None):
        ck = (UNROLL // 2) * j + (m >> 1)
        g_wait(m % NBUF)
        sc_start(ibuf, ck, m & 1, m % NBUF)
        if not skip_wait_sc:
            sc_wait((m - 1) % NBUF)
        if g_target is not None:
            ib3, ck3 = g_target
            g_start(ib3, ck3, (m + NBUF - 1) & 1, (m - 1) % NBUF)

    zc0 = pltpu.make_async_copy(
        zeros_hbm.at[pl.ds(s * ROWS_PER_SUB, ROWS_PER_SUB)],
        agg_sh.at[pl.ds(s * ROWS_PER_SUB, ROWS_PER_SUB)], zsem)
    zc0.start()
    for cp in idx_copies(0, 0):
        cp.start()

    @pl.when(s == NS - 1)
    def _():
        pltpu.sync_copy(
            zeros_hbm.at[pl.ds(NS * ROWS_PER_SUB, N - NS * ROWS_PER_SUB)],
            agg_sh.at[pl.ds(NS * ROWS_PER_SUB, N - NS * ROWS_PER_SUB)])

    for cp in idx_copies(0, 0):
        cp.wait()
    for cp in idx_copies(1, 1):
        cp.start()
    for t in range(NBUF - 1):
        g_start(0, t >> 1, t & 1, t)
    zc0.wait()
    plsc.subcore_barrier()

    HU = UNROLL // 2
    for blk in range(NBLK):
        ib = blk % 2
        for m in range(UNROLL):
            item(0, ib, m, skip_wait_sc=(blk == 0 and m == 0),
                 g_target=(ib, (m + NBUF - 1) >> 1))
        if 1 <= blk < NBLK - 1:
            for cp in idx_copies(blk + 1, 1 - ib):
                cp.start()

        @pl.loop(1, GRP - 1)
        def _(j, ib=ib):
            for m in range(UNROLL):
                item(j, ib, m, g_target=(ib, HU * j + ((m + NBUF - 1) >> 1)))

        jl = GRP - 1
        if blk + 1 < NBLK:
            nib = 1 - ib
            for cp in idx_copies(blk + 1, nib):
                cp.wait()
            for m in range(UNROLL):
                if m + NBUF - 1 < UNROLL:
                    tgt = (ib, HU * jl + ((m + NBUF - 1) >> 1))
                else:
                    tgt = (nib, (m + NBUF - 1 - UNROLL) >> 1)
                item(jl, ib, m, g_target=tgt)
        else:
            for m in range(UNROLL):
                tgt = None
                if m + NBUF - 1 < UNROLL:
                    tgt = (ib, HU * jl + ((m + NBUF - 1) >> 1))
                item(jl, ib, m, g_target=tgt)

    sc_wait((2 * NCHUNK - 1) % NBUF)
    plsc.subcore_barrier()
    pltpu.sync_copy(agg_sh.at[pl.ds(s * ROWS_PER_SUB, ROWS_PER_SUB)],
                    out_hbm.at[c].at[pl.ds(s * ROWS_PER_SUB, ROWS_PER_SUB)])

    @pl.when(s == NS - 1)
    def _():
        pltpu.sync_copy(
            agg_sh.at[pl.ds(NS * ROWS_PER_SUB, N - NS * ROWS_PER_SUB)],
            out_hbm.at[c].at[pl.ds(NS * ROWS_PER_SUB, N - NS * ROWS_PER_SUB)])


def _sc_agg(h1, src4, dst4, zeros):
    kfn = pl.kernel(
        _sc_agg_body,
        out_type=jax.ShapeDtypeStruct((NC, N, D), jnp.float32),
        mesh=plsc.VectorSubcoreMesh(core_axis_name="c", subcore_axis_name="s"),
        scratch_types=[
            pltpu.VMEM((2, BLK, B), jnp.int32),
            pltpu.VMEM((2, BLK, B), jnp.int32),
            pltpu.VMEM((NBUF, B, D), jnp.float32),
            pltpu.VMEM_SHARED((N, D), jnp.float32),
        ] + [pltpu.SemaphoreType.DMA] * (5 + 2 * NBUF),
    )
    return kfn(h1, src4, dst4, zeros)


BM = 1000


def _mm_body(f_ref, w_ref, b_ref, o_ref):
    o_ref[...] = (jnp.dot(f_ref[...], w_ref[...],
                          preferred_element_type=jnp.float32) + b_ref[...])


def _matmul(f, W, b):
    return pl.pallas_call(
        _mm_body,
        grid=(N // BM,),
        in_specs=[pl.BlockSpec((BM, D), lambda i: (i, 0)),
                  pl.BlockSpec((D, D), lambda i: (0, 0)),
                  pl.BlockSpec((1, D), lambda i: (0, 0))],
        out_specs=pl.BlockSpec((BM, D), lambda i: (i, 0)),
        out_shape=jax.ShapeDtypeStruct((N, D), jnp.float32),
    )(f, W, b.reshape(1, D))


def _norm_body(h0_ref, agg_ref, g_ref, bet_ref, *rest, add_res):
    if add_res:
        res_ref, o_ref = rest
    else:
        (o_ref,) = rest
    h = h0_ref[...] + agg_ref[0] + agg_ref[1]
    mu = jnp.mean(h, axis=-1, keepdims=True)
    var = jnp.mean((h - mu) ** 2, axis=-1, keepdims=True)
    h = (h - mu) * lax.rsqrt(var + EPS) * g_ref[...] + bet_ref[...]
    if add_res:
        h = h + res_ref[...]
    o_ref[...] = jnp.maximum(h, 0.0)


def _norm(h0, agg, gamma, beta, res):
    add_res = res is not None
    in_specs = [pl.BlockSpec((BM, D), lambda i: (i, 0)),
                pl.BlockSpec((NC, BM, D), lambda i: (0, i, 0)),
                pl.BlockSpec((1, D), lambda i: (0, 0)),
                pl.BlockSpec((1, D), lambda i: (0, 0))]
    args = [h0, agg, gamma.reshape(1, D), beta.reshape(1, D)]
    if add_res:
        in_specs.append(pl.BlockSpec((BM, D), lambda i: (i, 0)))
        args.append(res)
    return pl.pallas_call(
        functools.partial(_norm_body, add_res=add_res),
        grid=(N // BM,),
        in_specs=in_specs,
        out_specs=pl.BlockSpec((BM, D), lambda i: (i, 0)),
        out_shape=jax.ShapeDtypeStruct((N, D), jnp.float32),
    )(*args)


def kernel(features, edges, W0s, b0s, W1s, b1s, gammas, betas):
    src4 = edges[:, 0].reshape(NC * NS, NBLK, BLK, B)
    dst4 = edges[:, 1].reshape(NC * NS, NBLK, BLK, B)
    zeros = jnp.zeros((N, D), jnp.float32)
    f = features
    for i in range(N_LAYERS):
        h1 = _matmul(f, W1s[i], b1s[i])
        h0 = _matmul(f, W0s[i], b0s[i])
        agg = _sc_agg(h1, src4, dst4, zeros)
        f = _norm(h0, agg, gammas[i], betas[i],
                  features if i == N_LAYERS - 1 else None)
    return f


# speedup vs baseline: 10.8885x; 1.0233x over previous
import functools

import jax
import jax.numpy as jnp
from jax import lax
from jax.experimental import pallas as pl
from jax.experimental.pallas import tpu as pltpu
from jax.experimental.pallas import tpu_sc as plsc

N = 10000
D = 128
E = 320000
N_LAYERS = 3
EPS = 1e-5

NC = 2
NS = 16
B = 40
NCHUNK = E // (NC * NS * B)
NBLK = 10
BLK = NCHUNK // NBLK
NBUF = 5
UNROLL = 10
GRP = 2 * BLK // UNROLL
ROWS_PER_SUB = 624


def _sc_agg_body(h1_hbm, src_hbm, dst_hbm, zeros_hbm, out_hbm,
                 src_v, dst_v, rows_v, agg_sh,
                 zsem, ssem0, ssem1, dsem0, dsem1,
                 g0, g1, g2, g3, g4, c0, c1, c2, c3, c4):
    c = lax.axis_index("c")
    s = lax.axis_index("s")
    w = c * NS + s
    gsems = (g0, g1, g2, g3, g4)
    csems = (c0, c1, c2, c3, c4)
    ssems = (ssem0, ssem1)
    dsems = (dsem0, dsem1)


    def idx_copies(blk, ibuf):
        return (pltpu.make_async_copy(src_hbm.at[w, blk], src_v.at[ibuf],
                                      ssems[ibuf]),
                pltpu.make_async_copy(dst_hbm.at[w, blk], dst_v.at[ibuf],
                                      dsems[ibuf]))

    def g_start(ibuf, ck, d, buf):
        idx = dst_v if d == 0 else src_v
        pltpu.make_async_copy(h1_hbm.at[idx.at[ibuf, ck]], rows_v.at[buf],
                              gsems[buf]).start()

    def g_wait(buf):
        pltpu.make_async_copy(h1_hbm.at[src_v.at[0, 0]], rows_v.at[buf],
                              gsems[buf]).wait()

    def sc_start(ibuf, ck, d, buf):
        idx = src_v if d == 0 else dst_v
        pltpu.make_async_copy(rows_v.at[buf], agg_sh.at[idx.at[ibuf, ck]],
                              csems[buf]).start(add=True)

    def sc_wait(buf):
        pltpu.make_async_copy(rows_v.at[buf], agg_sh.at[src_v.at[0, 0]],
                              csems[buf]).wait()

    def item(j, ibuf, m, skip_wait_sc=False, g_target=None):
        ck = (UNROLL // 2) * j + (m >> 1)
        g_wait(m % NBUF)
        sc_start(ibuf, ck, m & 1, m % NBUF)
        if not skip_wait_sc:
            sc_wait((m - 1) % NBUF)
        if g_target is not None:
            ib3, ck3 = g_target
            g_start(ib3, ck3, (m + NBUF - 1) & 1, (m - 1) % NBUF)

    zc0 = pltpu.make_async_copy(
        zeros_hbm.at[pl.ds(s * ROWS_PER_SUB, ROWS_PER_SUB)],
        agg_sh.at[pl.ds(s * ROWS_PER_SUB, ROWS_PER_SUB)], zsem)
    zc0.start()
    for cp in idx_copies(0, 0):
        cp.start()

    @pl.when(s == NS - 1)
    def _():
        pltpu.sync_copy(
            zeros_hbm.at[pl.ds(NS * ROWS_PER_SUB, N - NS * ROWS_PER_SUB)],
            agg_sh.at[pl.ds(NS * ROWS_PER_SUB, N - NS * ROWS_PER_SUB)])

    for cp in idx_copies(0, 0):
        cp.wait()
    for cp in idx_copies(1, 1):
        cp.start()
    for t in range(NBUF - 1):
        g_start(0, t >> 1, t & 1, t)
    zc0.wait()
    plsc.subcore_barrier()

    HU = UNROLL // 2
    for blk in range(NBLK):
        ib = blk % 2
        for m in range(UNROLL):
            item(0, ib, m, skip_wait_sc=(blk == 0 and m == 0),
                 g_target=(ib, (m + NBUF - 1) >> 1))
        if 1 <= blk < NBLK - 1:
            for cp in idx_copies(blk + 1, 1 - ib):
                cp.start()

        @pl.loop(1, GRP - 1)
        def _(j, ib=ib):
            for m in range(UNROLL):
                item(j, ib, m, g_target=(ib, HU * j + ((m + NBUF - 1) >> 1)))

        jl = GRP - 1
        if blk + 1 < NBLK:
            nib = 1 - ib
            for cp in idx_copies(blk + 1, nib):
                cp.wait()
            for m in range(UNROLL):
                if m + NBUF - 1 < UNROLL:
                    tgt = (ib, HU * jl + ((m + NBUF - 1) >> 1))
                else:
                    tgt = (nib, (m + NBUF - 1 - UNROLL) >> 1)
                item(jl, ib, m, g_target=tgt)
        else:
            for m in range(UNROLL):
                tgt = None
                if m + NBUF - 1 < UNROLL:
                    tgt = (ib, HU * jl + ((m + NBUF - 1) >> 1))
                item(jl, ib, m, g_target=tgt)

    sc_wait((2 * NCHUNK - 1) % NBUF)
    plsc.subcore_barrier()
    pltpu.sync_copy(agg_sh.at[pl.ds(s * ROWS_PER_SUB, ROWS_PER_SUB)],
                    out_hbm.at[c].at[pl.ds(s * ROWS_PER_SUB, ROWS_PER_SUB)])

    @pl.when(s == NS - 1)
    def _():
        pltpu.sync_copy(
            agg_sh.at[pl.ds(NS * ROWS_PER_SUB, N - NS * ROWS_PER_SUB)],
            out_hbm.at[c].at[pl.ds(NS * ROWS_PER_SUB, N - NS * ROWS_PER_SUB)])


def _sc_agg(h1, src4, dst4, zeros):
    kfn = pl.kernel(
        _sc_agg_body,
        out_type=jax.ShapeDtypeStruct((NC, N, D), jnp.float32),
        mesh=plsc.VectorSubcoreMesh(core_axis_name="c", subcore_axis_name="s"),
        scratch_types=[
            pltpu.VMEM((2, BLK, B), jnp.int32),
            pltpu.VMEM((2, BLK, B), jnp.int32),
            pltpu.VMEM((NBUF, B, D), jnp.float32),
            pltpu.VMEM_SHARED((N, D), jnp.float32),
        ] + [pltpu.SemaphoreType.DMA] * (5 + 2 * NBUF),
    )
    return kfn(h1, src4, dst4, zeros)


BM = 1000


def _mm2_body(f_ref, w1_ref, b1_ref, w0_ref, b0_ref, o1_ref, o0_ref):
    f = f_ref[...]
    o1_ref[...] = (jnp.dot(f, w1_ref[...],
                           preferred_element_type=jnp.float32) + b1_ref[...])
    o0_ref[...] = (jnp.dot(f, w0_ref[...],
                           preferred_element_type=jnp.float32) + b0_ref[...])


def _mm2(f, W1, b1, W0, b0):
    return pl.pallas_call(
        _mm2_body,
        grid=(N // BM,),
        in_specs=[pl.BlockSpec((BM, D), lambda i: (i, 0)),
                  pl.BlockSpec((D, D), lambda i: (0, 0)),
                  pl.BlockSpec((1, D), lambda i: (0, 0)),
                  pl.BlockSpec((D, D), lambda i: (0, 0)),
                  pl.BlockSpec((1, D), lambda i: (0, 0))],
        out_specs=[pl.BlockSpec((BM, D), lambda i: (i, 0)),
                   pl.BlockSpec((BM, D), lambda i: (i, 0))],
        out_shape=[jax.ShapeDtypeStruct((N, D), jnp.float32),
                   jax.ShapeDtypeStruct((N, D), jnp.float32)],
    )(f, W1, b1.reshape(1, D), W0, b0.reshape(1, D))


def _mid_body(h0_ref, agg_ref, g_ref, bet_ref, w1_ref, b1_ref, w0_ref,
              b0_ref, o1_ref, o0_ref):
    h = h0_ref[...] + agg_ref[0] + agg_ref[1]
    mu = jnp.mean(h, axis=-1, keepdims=True)
    var = jnp.mean((h - mu) ** 2, axis=-1, keepdims=True)
    h = (h - mu) * lax.rsqrt(var + EPS) * g_ref[...] + bet_ref[...]
    f = jnp.maximum(h, 0.0)
    o1_ref[...] = (jnp.dot(f, w1_ref[...],
                           preferred_element_type=jnp.float32) + b1_ref[...])
    o0_ref[...] = (jnp.dot(f, w0_ref[...],
                           preferred_element_type=jnp.float32) + b0_ref[...])


def _mid(h0, agg, gamma, beta, W1, b1, W0, b0):
    return pl.pallas_call(
        _mid_body,
        grid=(N // BM,),
        in_specs=[pl.BlockSpec((BM, D), lambda i: (i, 0)),
                  pl.BlockSpec((NC, BM, D), lambda i: (0, i, 0)),
                  pl.BlockSpec((1, D), lambda i: (0, 0)),
                  pl.BlockSpec((1, D), lambda i: (0, 0)),
                  pl.BlockSpec((D, D), lambda i: (0, 0)),
                  pl.BlockSpec((1, D), lambda i: (0, 0)),
                  pl.BlockSpec((D, D), lambda i: (0, 0)),
                  pl.BlockSpec((1, D), lambda i: (0, 0))],
        out_specs=[pl.BlockSpec((BM, D), lambda i: (i, 0)),
                   pl.BlockSpec((BM, D), lambda i: (i, 0))],
        out_shape=[jax.ShapeDtypeStruct((N, D), jnp.float32),
                   jax.ShapeDtypeStruct((N, D), jnp.float32)],
    )(h0, agg, gamma.reshape(1, D), beta.reshape(1, D),
      W1, b1.reshape(1, D), W0, b0.reshape(1, D))


def _norm_body(h0_ref, agg_ref, g_ref, bet_ref, *rest, add_res):
    if add_res:
        res_ref, o_ref = rest
    else:
        (o_ref,) = rest
    h = h0_ref[...] + agg_ref[0] + agg_ref[1]
    mu = jnp.mean(h, axis=-1, keepdims=True)
    var = jnp.mean((h - mu) ** 2, axis=-1, keepdims=True)
    h = (h - mu) * lax.rsqrt(var + EPS) * g_ref[...] + bet_ref[...]
    if add_res:
        h = h + res_ref[...]
    o_ref[...] = jnp.maximum(h, 0.0)


def _norm(h0, agg, gamma, beta, res):
    add_res = res is not None
    in_specs = [pl.BlockSpec((BM, D), lambda i: (i, 0)),
                pl.BlockSpec((NC, BM, D), lambda i: (0, i, 0)),
                pl.BlockSpec((1, D), lambda i: (0, 0)),
                pl.BlockSpec((1, D), lambda i: (0, 0))]
    args = [h0, agg, gamma.reshape(1, D), beta.reshape(1, D)]
    if add_res:
        in_specs.append(pl.BlockSpec((BM, D), lambda i: (i, 0)))
        args.append(res)
    return pl.pallas_call(
        functools.partial(_norm_body, add_res=add_res),
        grid=(N // BM,),
        in_specs=in_specs,
        out_specs=pl.BlockSpec((BM, D), lambda i: (i, 0)),
        out_shape=jax.ShapeDtypeStruct((N, D), jnp.float32),
    )(*args)


def kernel(features, edges, W0s, b0s, W1s, b1s, gammas, betas):
    src4 = edges[:, 0].reshape(NC * NS, NBLK, BLK, B)
    dst4 = edges[:, 1].reshape(NC * NS, NBLK, BLK, B)
    zeros = jnp.zeros((N, D), jnp.float32)
    h1, h0 = _mm2(features, W1s[0], b1s[0], W0s[0], b0s[0])
    for i in range(N_LAYERS):
        agg = _sc_agg(h1, src4, dst4, zeros)
        if i < N_LAYERS - 1:
            h1, h0 = _mid(h0, agg, gammas[i], betas[i],
                          W1s[i + 1], b1s[i + 1], W0s[i + 1], b0s[i + 1])
        else:
            f = _norm(h0, agg, gammas[i], betas[i], features)
    return f


# speedup vs baseline: 10.9029x; 1.0013x over previous
import functools

import jax
import jax.numpy as jnp
from jax import lax
from jax.experimental import pallas as pl
from jax.experimental.pallas import tpu as pltpu
from jax.experimental.pallas import tpu_sc as plsc

N = 10000
D = 128
E = 320000
N_LAYERS = 3
EPS = 1e-5

NC = 2
NS = 16
B = 40
NCHUNK = E // (NC * NS * B)
NBLK = 10
BLK = NCHUNK // NBLK
NBUF = 5
UNROLL = 10
GRP = 2 * BLK // UNROLL
ROWS_PER_SUB = 624


def _sc_agg_body(h1_hbm, src_hbm, dst_hbm, zeros_hbm, out_hbm,
                 src_v, dst_v, rows_v, agg_sh,
                 zsem, ssem0, ssem1, dsem0, dsem1,
                 g0, g1, g2, g3, g4, c0, c1, c2, c3, c4):
    c = lax.axis_index("c")
    s = lax.axis_index("s")
    w = c * NS + s
    gsems = (g0, g1, g2, g3, g4)
    csems = (c0, c1, c2, c3, c4)
    ssems = (ssem0, ssem1)
    dsems = (dsem0, dsem1)


    def idx_copies(blk, ibuf):
        return (pltpu.make_async_copy(src_hbm.at[w, blk], src_v.at[ibuf],
                                      ssems[ibuf]),
                pltpu.make_async_copy(dst_hbm.at[w, blk], dst_v.at[ibuf],
                                      dsems[ibuf]))

    def g_start(ibuf, ck, d, buf):
        idx = dst_v if d == 0 else src_v
        pltpu.make_async_copy(h1_hbm.at[idx.at[ibuf, ck]], rows_v.at[buf],
                              gsems[buf]).start()

    def g_wait(buf):
        pltpu.make_async_copy(h1_hbm.at[src_v.at[0, 0]], rows_v.at[buf],
                              gsems[buf]).wait()

    def sc_start(ibuf, ck, d, buf):
        idx = src_v if d == 0 else dst_v
        pltpu.make_async_copy(rows_v.at[buf], agg_sh.at[idx.at[ibuf, ck]],
                              csems[buf]).start(add=True)

    def sc_wait(buf):
        pltpu.make_async_copy(rows_v.at[buf], agg_sh.at[src_v.at[0, 0]],
                              csems[buf]).wait()

    def item(j, ibuf, m, skip_wait_sc=False, g_target=None):
        ck = (UNROLL // 2) * j + (m >> 1)
        g_wait(m % NBUF)
        sc_start(ibuf, ck, m & 1, m % NBUF)
        if not skip_wait_sc:
            sc_wait((m - 1) % NBUF)
        if g_target is not None:
            ib3, ck3 = g_target
            g_start(ib3, ck3, (m + NBUF - 1) & 1, (m - 1) % NBUF)

    zc0 = pltpu.make_async_copy(
        zeros_hbm.at[pl.ds(s * ROWS_PER_SUB, ROWS_PER_SUB)],
        agg_sh.at[pl.ds(s * ROWS_PER_SUB, ROWS_PER_SUB)], zsem)
    zc0.start()
    for cp in idx_copies(0, 0):
        cp.start()

    @pl.when(s == NS - 1)
    def _():
        pltpu.sync_copy(
            zeros_hbm.at[pl.ds(NS * ROWS_PER_SUB, N - NS * ROWS_PER_SUB)],
            agg_sh.at[pl.ds(NS * ROWS_PER_SUB, N - NS * ROWS_PER_SUB)])

    for cp in idx_copies(0, 0):
        cp.wait()
    for cp in idx_copies(1, 1):
        cp.start()
    for t in range(NBUF - 1):
        g_start(0, t >> 1, t & 1, t)
    zc0.wait()
    plsc.subcore_barrier()

    HU = UNROLL // 2
    for blk in range(NBLK):
        ib = blk % 2
        for m in range(UNROLL):
            item(0, ib, m, skip_wait_sc=(blk == 0 and m == 0),
                 g_target=(ib, (m + NBUF - 1) >> 1))
        if 1 <= blk < NBLK - 1:
            for cp in idx_copies(blk + 1, 1 - ib):
                cp.start()

        @pl.loop(1, GRP - 1)
        def _(j, ib=ib):
            for m in range(UNROLL):
                item(j, ib, m, g_target=(ib, HU * j + ((m + NBUF - 1) >> 1)))

        jl = GRP - 1
        if blk + 1 < NBLK:
            nib = 1 - ib
            for cp in idx_copies(blk + 1, nib):
                cp.wait()
            for m in range(UNROLL):
                if m + NBUF - 1 < UNROLL:
                    tgt = (ib, HU * jl + ((m + NBUF - 1) >> 1))
                else:
                    tgt = (nib, (m + NBUF - 1 - UNROLL) >> 1)
                item(jl, ib, m, g_target=tgt)
        else:
            for m in range(UNROLL):
                tgt = None
                if m + NBUF - 1 < UNROLL:
                    tgt = (ib, HU * jl + ((m + NBUF - 1) >> 1))
                item(jl, ib, m, g_target=tgt)

    sc_wait((2 * NCHUNK - 1) % NBUF)
    plsc.subcore_barrier()
    pltpu.sync_copy(agg_sh.at[pl.ds(s * ROWS_PER_SUB, ROWS_PER_SUB)],
                    out_hbm.at[c].at[pl.ds(s * ROWS_PER_SUB, ROWS_PER_SUB)])

    @pl.when(s == NS - 1)
    def _():
        pltpu.sync_copy(
            agg_sh.at[pl.ds(NS * ROWS_PER_SUB, N - NS * ROWS_PER_SUB)],
            out_hbm.at[c].at[pl.ds(NS * ROWS_PER_SUB, N - NS * ROWS_PER_SUB)])


def _sc_agg(h1, src4, dst4, zeros):
    kfn = pl.kernel(
        _sc_agg_body,
        out_type=jax.ShapeDtypeStruct((NC, N, D), jnp.float32),
        mesh=plsc.VectorSubcoreMesh(core_axis_name="c", subcore_axis_name="s"),
        scratch_types=[
            pltpu.VMEM((2, BLK, B), jnp.int32),
            pltpu.VMEM((2, BLK, B), jnp.int32),
            pltpu.VMEM((NBUF, B, D), jnp.float32),
            pltpu.VMEM_SHARED((N, D), jnp.float32),
        ] + [pltpu.SemaphoreType.DMA] * (5 + 2 * NBUF),
    )
    return kfn(h1, src4, dst4, zeros)


BM = 1000


def _mm2_body(f_ref, w1_ref, b1_ref, w0_ref, b0_ref, o1_ref, o0_ref):
    f = f_ref[...]
    o1_ref[...] = (jnp.dot(f, w1_ref[...],
                           preferred_element_type=jnp.float32) + b1_ref[...])
    o0_ref[...] = (jnp.dot(f, w0_ref[...],
                           preferred_element_type=jnp.float32) + b0_ref[...])


def _mm2(f, W1, b1, W0, b0):
    return pl.pallas_call(
        _mm2_body,
        grid=(N // BM,),
        in_specs=[pl.BlockSpec((BM, D), lambda i: (i, 0)),
                  pl.BlockSpec((D, D), lambda i: (0, 0)),
                  pl.BlockSpec((1, D), lambda i: (0, 0)),
                  pl.BlockSpec((D, D), lambda i: (0, 0)),
                  pl.BlockSpec((1, D), lambda i: (0, 0))],
        out_specs=[pl.BlockSpec((BM, D), lambda i: (i, 0)),
                   pl.BlockSpec((BM, D), lambda i: (i, 0))],
        out_shape=[jax.ShapeDtypeStruct((N, D), jnp.float32),
                   jax.ShapeDtypeStruct((N, D), jnp.float32)],
    )(f, W1, b1.reshape(1, D), W0, b0.reshape(1, D))


def _mid_body(h0_ref, agg_ref, g_ref, bet_ref, w1_ref, b1_ref,
              o1_ref, of_ref):
    h = h0_ref[...] + agg_ref[0] + agg_ref[1]
    mu = jnp.mean(h, axis=-1, keepdims=True)
    var = jnp.mean((h - mu) ** 2, axis=-1, keepdims=True)
    h = (h - mu) * lax.rsqrt(var + EPS) * g_ref[...] + bet_ref[...]
    f = jnp.maximum(h, 0.0)
    of_ref[...] = f
    o1_ref[...] = (jnp.dot(f, w1_ref[...],
                           preferred_element_type=jnp.float32) + b1_ref[...])


def _mid(h0, agg, gamma, beta, W1, b1):
    return pl.pallas_call(
        _mid_body,
        grid=(N // BM,),
        in_specs=[pl.BlockSpec((BM, D), lambda i: (i, 0)),
                  pl.BlockSpec((NC, BM, D), lambda i: (0, i, 0)),
                  pl.BlockSpec((1, D), lambda i: (0, 0)),
                  pl.BlockSpec((1, D), lambda i: (0, 0)),
                  pl.BlockSpec((D, D), lambda i: (0, 0)),
                  pl.BlockSpec((1, D), lambda i: (0, 0))],
        out_specs=[pl.BlockSpec((BM, D), lambda i: (i, 0)),
                   pl.BlockSpec((BM, D), lambda i: (i, 0))],
        out_shape=[jax.ShapeDtypeStruct((N, D), jnp.float32),
                   jax.ShapeDtypeStruct((N, D), jnp.float32)],
    )(h0, agg, gamma.reshape(1, D), beta.reshape(1, D),
      W1, b1.reshape(1, D))


def _mm_body(f_ref, w_ref, b_ref, o_ref):
    o_ref[...] = (jnp.dot(f_ref[...], w_ref[...],
                          preferred_element_type=jnp.float32) + b_ref[...])


def _matmul(f, W, b):
    return pl.pallas_call(
        _mm_body,
        grid=(N // BM,),
        in_specs=[pl.BlockSpec((BM, D), lambda i: (i, 0)),
                  pl.BlockSpec((D, D), lambda i: (0, 0)),
                  pl.BlockSpec((1, D), lambda i: (0, 0))],
        out_specs=pl.BlockSpec((BM, D), lambda i: (i, 0)),
        out_shape=jax.ShapeDtypeStruct((N, D), jnp.float32),
    )(f, W, b.reshape(1, D))


def _norm_body(h0_ref, agg_ref, g_ref, bet_ref, *rest, add_res):
    if add_res:
        res_ref, o_ref = rest
    else:
        (o_ref,) = rest
    h = h0_ref[...] + agg_ref[0] + agg_ref[1]
    mu = jnp.mean(h, axis=-1, keepdims=True)
    var = jnp.mean((h - mu) ** 2, axis=-1, keepdims=True)
    h = (h - mu) * lax.rsqrt(var + EPS) * g_ref[...] + bet_ref[...]
    if add_res:
        h = h + res_ref[...]
    o_ref[...] = jnp.maximum(h, 0.0)


def _norm(h0, agg, gamma, beta, res):
    add_res = res is not None
    in_specs = [pl.BlockSpec((BM, D), lambda i: (i, 0)),
                pl.BlockSpec((NC, BM, D), lambda i: (0, i, 0)),
                pl.BlockSpec((1, D), lambda i: (0, 0)),
                pl.BlockSpec((1, D), lambda i: (0, 0))]
    args = [h0, agg, gamma.reshape(1, D), beta.reshape(1, D)]
    if add_res:
        in_specs.append(pl.BlockSpec((BM, D), lambda i: (i, 0)))
        args.append(res)
    return pl.pallas_call(
        functools.partial(_norm_body, add_res=add_res),
        grid=(N // BM,),
        in_specs=in_specs,
        out_specs=pl.BlockSpec((BM, D), lambda i: (i, 0)),
        out_shape=jax.ShapeDtypeStruct((N, D), jnp.float32),
    )(*args)


def kernel(features, edges, W0s, b0s, W1s, b1s, gammas, betas):
    src4 = edges[:, 0].reshape(NC * NS, NBLK, BLK, B)
    dst4 = edges[:, 1].reshape(NC * NS, NBLK, BLK, B)
    zeros = jnp.zeros((N, D), jnp.float32)
    h1, h0 = _mm2(features, W1s[0], b1s[0], W0s[0], b0s[0])
    for i in range(N_LAYERS):
        agg = _sc_agg(h1, src4, dst4, zeros)
        if i < N_LAYERS - 1:
            h1, f = _mid(h0, agg, gammas[i], betas[i],
                         W1s[i + 1], b1s[i + 1])
            h0 = _matmul(f, W0s[i + 1], b0s[i + 1])
        else:
            f = _norm(h0, agg, gammas[i], betas[i], features)
    return f


# speedup vs baseline: 11.0443x; 1.0130x over previous
import functools

import jax
import jax.numpy as jnp
from jax import lax
from jax.experimental import pallas as pl
from jax.experimental.pallas import tpu as pltpu
from jax.experimental.pallas import tpu_sc as plsc

N = 10000
D = 128
E = 320000
N_LAYERS = 3
EPS = 1e-5

NC = 2
NS = 16
B = 40
NCHUNK = E // (NC * NS * B)
NBLK = 10
BLK = NCHUNK // NBLK
NBUF = 5
UNROLL = 10
GRP = 2 * BLK // UNROLL
ROWS_PER_SUB = 624


def _sc_agg_body(h1_hbm, src_hbm, dst_hbm, zeros_hbm, out_hbm,
                 src_v, dst_v, rows_v, agg_sh,
                 zsem, ssem0, ssem1, dsem0, dsem1,
                 g0, g1, g2, g3, g4, c0, c1, c2, c3, c4):
    c = lax.axis_index("c")
    s = lax.axis_index("s")
    w = c * NS + s
    gsems = (g0, g1, g2, g3, g4)
    csems = (c0, c1, c2, c3, c4)
    ssems = (ssem0, ssem1)
    dsems = (dsem0, dsem1)


    def idx_copies(blk, ibuf):
        return (pltpu.make_async_copy(src_hbm.at[w, blk], src_v.at[ibuf],
                                      ssems[ibuf]),
                pltpu.make_async_copy(dst_hbm.at[w, blk], dst_v.at[ibuf],
                                      dsems[ibuf]))

    def g_start(ibuf, ck, d, buf):
        idx = dst_v if d == 0 else src_v
        pltpu.make_async_copy(h1_hbm.at[idx.at[ibuf, ck]], rows_v.at[buf],
                              gsems[buf]).start()

    def g_wait(buf):
        pltpu.make_async_copy(h1_hbm.at[src_v.at[0, 0]], rows_v.at[buf],
                              gsems[buf]).wait()

    def sc_start(ibuf, ck, d, buf):
        idx = src_v if d == 0 else dst_v
        pltpu.make_async_copy(rows_v.at[buf], agg_sh.at[idx.at[ibuf, ck]],
                              csems[buf]).start(add=True)

    def sc_wait(buf):
        pltpu.make_async_copy(rows_v.at[buf], agg_sh.at[src_v.at[0, 0]],
                              csems[buf]).wait()

    def item(j, ibuf, m, skip_wait_sc=False, g_target=None):
        ck = (UNROLL // 2) * j + (m >> 1)
        g_wait(m % NBUF)
        sc_start(ibuf, ck, m & 1, m % NBUF)
        if not skip_wait_sc:
            sc_wait((m - 1) % NBUF)
        if g_target is not None:
            ib3, ck3 = g_target
            g_start(ib3, ck3, (m + NBUF - 1) & 1, (m - 1) % NBUF)

    zc0 = pltpu.make_async_copy(
        zeros_hbm.at[pl.ds(s * ROWS_PER_SUB, ROWS_PER_SUB)],
        agg_sh.at[pl.ds(s * ROWS_PER_SUB, ROWS_PER_SUB)], zsem)
    zc0.start()
    for cp in idx_copies(0, 0):
        cp.start()

    @pl.when(s == NS - 1)
    def _():
        pltpu.sync_copy(
            zeros_hbm.at[pl.ds(NS * ROWS_PER_SUB, N - NS * ROWS_PER_SUB)],
            agg_sh.at[pl.ds(NS * ROWS_PER_SUB, N - NS * ROWS_PER_SUB)])

    for cp in idx_copies(0, 0):
        cp.wait()
    for cp in idx_copies(1, 1):
        cp.start()
    for t in range(NBUF - 1):
        g_start(0, t >> 1, t & 1, t)
    zc0.wait()
    plsc.subcore_barrier()

    HU = UNROLL // 2
    for blk in range(NBLK):
        ib = blk % 2
        for m in range(UNROLL):
            item(0, ib, m, skip_wait_sc=(blk == 0 and m == 0),
                 g_target=(ib, (m + NBUF - 1) >> 1))
        if 1 <= blk < NBLK - 1:
            for cp in idx_copies(blk + 1, 1 - ib):
                cp.start()

        @pl.loop(1, GRP - 1)
        def _(j, ib=ib):
            for m in range(UNROLL):
                item(j, ib, m, g_target=(ib, HU * j + ((m + NBUF - 1) >> 1)))

        jl = GRP - 1
        if blk + 1 < NBLK:
            nib = 1 - ib
            for cp in idx_copies(blk + 1, nib):
                cp.wait()
            for m in range(UNROLL):
                if m + NBUF - 1 < UNROLL:
                    tgt = (ib, HU * jl + ((m + NBUF - 1) >> 1))
                else:
                    tgt = (nib, (m + NBUF - 1 - UNROLL) >> 1)
                item(jl, ib, m, g_target=tgt)
        else:
            for m in range(UNROLL):
                tgt = None
                if m + NBUF - 1 < UNROLL:
                    tgt = (ib, HU * jl + ((m + NBUF - 1) >> 1))
                item(jl, ib, m, g_target=tgt)

    sc_wait((2 * NCHUNK - 1) % NBUF)
    plsc.subcore_barrier()
    pltpu.sync_copy(agg_sh.at[pl.ds(s * ROWS_PER_SUB, ROWS_PER_SUB)],
                    out_hbm.at[c].at[pl.ds(s * ROWS_PER_SUB, ROWS_PER_SUB)])

    @pl.when(s == NS - 1)
    def _():
        pltpu.sync_copy(
            agg_sh.at[pl.ds(NS * ROWS_PER_SUB, N - NS * ROWS_PER_SUB)],
            out_hbm.at[c].at[pl.ds(NS * ROWS_PER_SUB, N - NS * ROWS_PER_SUB)])


def _sc_agg(h1, src4, dst4, zeros):
    kfn = pl.kernel(
        _sc_agg_body,
        out_type=jax.ShapeDtypeStruct((NC, N, D), jnp.float32),
        mesh=plsc.VectorSubcoreMesh(core_axis_name="c", subcore_axis_name="s"),
        scratch_types=[
            pltpu.VMEM((2, BLK, B), jnp.int32),
            pltpu.VMEM((2, BLK, B), jnp.int32),
            pltpu.VMEM((NBUF, B, D), jnp.float32),
            pltpu.VMEM_SHARED((N, D), jnp.float32),
        ] + [pltpu.SemaphoreType.DMA] * (5 + 2 * NBUF),
    )
    return kfn(h1, src4, dst4, zeros)


BM = 2000


def _mid_body(h0_ref, agg_ref, g_ref, bet_ref, w1_ref, b1_ref,
              o1_ref, of_ref):
    h = h0_ref[...] + agg_ref[0] + agg_ref[1]
    mu = jnp.mean(h, axis=-1, keepdims=True)
    var = jnp.mean((h - mu) ** 2, axis=-1, keepdims=True)
    h = (h - mu) * lax.rsqrt(var + EPS) * g_ref[...] + bet_ref[...]
    f = jnp.maximum(h, 0.0)
    of_ref[...] = f
    o1_ref[...] = (jnp.dot(f, w1_ref[...],
                           preferred_element_type=jnp.float32) + b1_ref[...])


def _mid(h0, agg, gamma, beta, W1, b1):
    return pl.pallas_call(
        _mid_body,
        grid=(N // BM,),
        in_specs=[pl.BlockSpec((BM, D), lambda i: (i, 0)),
                  pl.BlockSpec((NC, BM, D), lambda i: (0, i, 0)),
                  pl.BlockSpec((1, D), lambda i: (0, 0)),
                  pl.BlockSpec((1, D), lambda i: (0, 0)),
                  pl.BlockSpec((D, D), lambda i: (0, 0)),
                  pl.BlockSpec((1, D), lambda i: (0, 0))],
        out_specs=[pl.BlockSpec((BM, D), lambda i: (i, 0)),
                   pl.BlockSpec((BM, D), lambda i: (i, 0))],
        out_shape=[jax.ShapeDtypeStruct((N, D), jnp.float32),
                   jax.ShapeDtypeStruct((N, D), jnp.float32)],
    )(h0, agg, gamma.reshape(1, D), beta.reshape(1, D),
      W1, b1.reshape(1, D))


def _mm_body(f_ref, w_ref, b_ref, o_ref):
    o_ref[...] = (jnp.dot(f_ref[...], w_ref[...],
                          preferred_element_type=jnp.float32) + b_ref[...])


def _matmul(f, W, b):
    return pl.pallas_call(
        _mm_body,
        grid=(N // BM,),
        in_specs=[pl.BlockSpec((BM, D), lambda i: (i, 0)),
                  pl.BlockSpec((D, D), lambda i: (0, 0)),
                  pl.BlockSpec((1, D), lambda i: (0, 0))],
        out_specs=pl.BlockSpec((BM, D), lambda i: (i, 0)),
        out_shape=jax.ShapeDtypeStruct((N, D), jnp.float32),
    )(f, W, b.reshape(1, D))


def _norm_body(h0_ref, agg_ref, g_ref, bet_ref, *rest, add_res):
    if add_res:
        res_ref, o_ref = rest
    else:
        (o_ref,) = rest
    h = h0_ref[...] + agg_ref[0] + agg_ref[1]
    mu = jnp.mean(h, axis=-1, keepdims=True)
    var = jnp.mean((h - mu) ** 2, axis=-1, keepdims=True)
    h = (h - mu) * lax.rsqrt(var + EPS) * g_ref[...] + bet_ref[...]
    if add_res:
        h = h + res_ref[...]
    o_ref[...] = jnp.maximum(h, 0.0)


def _norm(h0, agg, gamma, beta, res):
    add_res = res is not None
    in_specs = [pl.BlockSpec((BM, D), lambda i: (i, 0)),
                pl.BlockSpec((NC, BM, D), lambda i: (0, i, 0)),
                pl.BlockSpec((1, D), lambda i: (0, 0)),
                pl.BlockSpec((1, D), lambda i: (0, 0))]
    args = [h0, agg, gamma.reshape(1, D), beta.reshape(1, D)]
    if add_res:
        in_specs.append(pl.BlockSpec((BM, D), lambda i: (i, 0)))
        args.append(res)
    return pl.pallas_call(
        functools.partial(_norm_body, add_res=add_res),
        grid=(N // BM,),
        in_specs=in_specs,
        out_specs=pl.BlockSpec((BM, D), lambda i: (i, 0)),
        out_shape=jax.ShapeDtypeStruct((N, D), jnp.float32),
    )(*args)


def kernel(features, edges, W0s, b0s, W1s, b1s, gammas, betas):
    src4 = edges[:, 0].reshape(NC * NS, NBLK, BLK, B)
    dst4 = edges[:, 1].reshape(NC * NS, NBLK, BLK, B)
    zeros = jnp.zeros((N, D), jnp.float32)
    h1 = _matmul(features, W1s[0], b1s[0])
    h0 = _matmul(features, W0s[0], b0s[0])
    for i in range(N_LAYERS):
        agg = _sc_agg(h1, src4, dst4, zeros)
        if i < N_LAYERS - 1:
            h1, f = _mid(h0, agg, gammas[i], betas[i],
                         W1s[i + 1], b1s[i + 1])
            h0 = _matmul(f, W0s[i + 1], b0s[i + 1])
        else:
            f = _norm(h0, agg, gammas[i], betas[i], features)
    return f
